# baseline (speedup 1.0000x reference)
import jax
import jax.numpy as jnp
from jax.experimental import pallas as pl
from jax.experimental.pallas import tpu as pltpu

_BC = 64


def _pos_kernel(col_ref, row_ref, out_ref):
    bc, h, w = out_ref.shape
    nb_half = pl.num_programs(0) // 2
    b = pl.program_id(0)
    for k in range(2 * nb_half):
        @pl.when(b == k)
        def _(k=k):
            if k < nb_half:
                slab = col_ref[0:w, :].T[k * bc:(k + 1) * bc, :]
                out_ref[...] = jnp.broadcast_to(slab[:, None, :], (bc, h, w))
            else:
                kk = k - nb_half
                slab = row_ref[0:h, :].T[kk * bc:(kk + 1) * bc, :]
                out_ref[...] = jnp.broadcast_to(slab[:, :, None], (bc, h, w))


def kernel(image_tensor, row_embed, col_embed):
    h, w = image_tensor.shape[-2], image_tensor.shape[-1]
    F = row_embed.shape[1]
    n_emb = row_embed.shape[0]
    out = pl.pallas_call(
        _pos_kernel,
        grid=(2 * F // _BC,),
        in_specs=[
            pl.BlockSpec((n_emb, F), lambda b: (0, 0)),
            pl.BlockSpec((n_emb, F), lambda b: (0, 0)),
        ],
        out_specs=pl.BlockSpec((_BC, h, w), lambda b: (b, 0, 0)),
        out_shape=jax.ShapeDtypeStruct((2 * F, h, w), jnp.float32),
        compiler_params=pltpu.CompilerParams(dimension_semantics=("parallel",)),
    )(col_embed, row_embed)
    return out[None]


# speedup vs baseline: 1.1164x; 1.1164x over previous
import jax
import jax.numpy as jnp
from jax.experimental import pallas as pl
from jax.experimental.pallas import tpu as pltpu

_NCP = 8


def _pos_kernel(col_ref, row_ref, out_ref, scratch, sems):
    c2, h, w = scratch.shape
    F = c2 // 2
    colT = col_ref[0:w, :].T
    rowT = row_ref[0:h, :].T
    blk = c2 // _NCP
    copies = []
    for k in range(_NCP):
        c0 = k * blk
        if c0 + blk <= F:
            slab = colT[c0:c0 + blk]
            scratch[c0:c0 + blk] = jnp.broadcast_to(slab[:, None, :], (blk, h, w))
        else:
            slab = rowT[c0 - F:c0 - F + blk]
            scratch[c0:c0 + blk] = jnp.broadcast_to(slab[:, :, None], (blk, h, w))
        cp = pltpu.make_async_copy(
            scratch.at[pl.ds(c0, blk)], out_ref.at[pl.ds(c0, blk)], sems.at[k]
        )
        cp.start()
        copies.append(cp)
    for cp in copies:
        cp.wait()


def kernel(image_tensor, row_embed, col_embed):
    h, w = image_tensor.shape[-2], image_tensor.shape[-1]
    F = row_embed.shape[1]
    out = pl.pallas_call(
        _pos_kernel,
        in_specs=[
            pl.BlockSpec(memory_space=pltpu.VMEM),
            pl.BlockSpec(memory_space=pltpu.VMEM),
        ],
        out_specs=pl.BlockSpec(memory_space=pl.ANY),
        out_shape=jax.ShapeDtypeStruct((2 * F, h, w), jnp.float32),
        scratch_shapes=[
            pltpu.VMEM((2 * F, h, w), jnp.float32),
            pltpu.SemaphoreType.DMA((_NCP,)),
        ],
    )(col_embed, row_embed)
    return out[None]
